# baseline (speedup 1.0000x reference)
import jax
import jax.numpy as jnp
from jax.experimental import pallas as pl
from jax.experimental.pallas import tpu as pltpu

_BATCH = 4


def _scale_kernel(stage_ref, emb_t_ref, x_ref, o_ref):
    s = stage_ref[0]
    emb_t = emb_t_ref[...]
    col = jax.lax.broadcasted_iota(jnp.int32, emb_t.shape, 1)
    scale = jnp.sum(jnp.where(col == s, emb_t, 0.0), axis=1)
    o_ref[...] = x_ref[...] * scale[None, :, None]


def kernel(x, stage, embeddings):
    b, c, h, w = x.shape
    x3 = x.reshape(b, c, h * w)
    stage_arr = jnp.asarray(stage, jnp.int32).reshape((1,))
    emb_t = embeddings.T

    out = pl.pallas_call(
        _scale_kernel,
        grid=(b // _BATCH,),
        in_specs=[
            pl.BlockSpec(memory_space=pltpu.SMEM),
            pl.BlockSpec((c, embeddings.shape[0]), lambda i: (0, 0)),
            pl.BlockSpec((_BATCH, c, h * w), lambda i: (i, 0, 0)),
        ],
        out_specs=pl.BlockSpec((_BATCH, c, h * w), lambda i: (i, 0, 0)),
        out_shape=jax.ShapeDtypeStruct((b, c, h * w), x.dtype),
        compiler_params=pltpu.CompilerParams(
            dimension_semantics=("arbitrary",),
        ),
    )(stage_arr, emb_t, x3)
    return out.reshape(b, c, h, w)
